# baseline (speedup 1.0000x reference)
import functools

import jax
import jax.numpy as jnp
from jax import lax
from jax.experimental import pallas as pl
from jax.experimental.pallas import tpu as pltpu
from jax.experimental.pallas import tpu_sc as plsc

_NSC = 2
_NTEC = 16
_NW = _NSC * _NTEC


def _mesh():
    return plsc.VectorSubcoreMesh(core_axis_name="c", subcore_axis_name="s")


@functools.lru_cache(maxsize=None)
def _sc_gather_rows(E, D):
    B = 64
    NB = E // B

    @functools.partial(
        pl.kernel,
        out_type=jax.ShapeDtypeStruct((E, D), jnp.float32),
        mesh=_mesh(),
        scratch_types=[
            pltpu.VMEM((B,), jnp.int32),
            pltpu.VMEM((B, D), jnp.float32),
            pltpu.SemaphoreType.DMA,
        ],
    )
    def k(table, idx, out, idx_v, buf, sem):
        c = lax.axis_index("c")
        s = lax.axis_index("s")
        w = s * _NSC + c
        lo = (NB * w) // _NW
        hi = (NB * (w + 1)) // _NW

        def body(r, carry):
            pltpu.sync_copy(idx.at[pl.ds(r * B, B)], idx_v)
            pltpu.async_copy(table.at[idx_v], buf, sem).wait()
            pltpu.sync_copy(buf, out.at[pl.ds(r * B, B)])
            return carry

        lax.fori_loop(lo, hi, body, 0)

    return k


@functools.lru_cache(maxsize=None)
def _sc_segsum(E, N, H):
    B = 128
    NB = E // B
    NCH = H // 128
    CPS = NCH // _NSC
    ZR = 40
    NGB = N // ZR

    @functools.partial(
        pl.kernel,
        out_type=jax.ShapeDtypeStruct((N, H), jnp.float32),
        mesh=_mesh(),
        scratch_types=[
            pltpu.VMEM((B, 128), jnp.float32),
            pltpu.VMEM((B,), jnp.int32),
            pltpu.VMEM((ZR, 128), jnp.float32),
            pltpu.VMEM((ZR, 128), jnp.float32),
            pltpu.VMEM_SHARED((N, 128), jnp.float32),
        ],
    )
    def k(bh, tgt, out, data_v, tidx, zbuf, wbuf, table):
        c = lax.axis_index("c")
        s = lax.axis_index("s")
        zero = jnp.zeros((16,), jnp.float32)

        def zrow(i, carry):
            for j in range(8):
                zbuf[i, pl.ds(j * 16, 16)] = zero
            return carry

        lax.fori_loop(0, ZR, zrow, 0)

        lo = (NB * s) // _NTEC
        hi = (NB * (s + 1)) // _NTEC
        glo = (NGB * s) // _NTEC
        ghi = (NGB * (s + 1)) // _NTEC

        for ci in range(CPS):
            fo = (c * CPS + ci) * 128

            def zblk(g, carry):
                pltpu.sync_copy(zbuf, table.at[pl.ds(g * ZR, ZR)])
                return carry

            lax.fori_loop(glo, ghi, zblk, 0)
            plsc.subcore_barrier()

            def body(r, carry):
                pltpu.sync_copy(tgt.at[pl.ds(r * B, B)], tidx)
                pltpu.sync_copy(bh.at[pl.ds(r * B, B), pl.ds(fo, 128)], data_v)
                pltpu.sync_copy(data_v, table.at[tidx], add=True)
                return carry

            lax.fori_loop(lo, hi, body, 0)
            plsc.subcore_barrier()

            def wblk(g, carry):
                pltpu.sync_copy(table.at[pl.ds(g * ZR, ZR)], wbuf)
                pltpu.sync_copy(wbuf, out.at[pl.ds(g * ZR, ZR), pl.ds(fo, 128)])
                return carry

            lax.fori_loop(glo, ghi, wblk, 0)
            plsc.subcore_barrier()

    return k


@functools.lru_cache(maxsize=None)
def _sc_agg(E, EP, N, H):
    B = 64
    NB = E // B
    CAP = NB // _NW + 1

    @functools.partial(
        pl.kernel,
        out_type=jax.ShapeDtypeStruct((E, H), jnp.float32),
        mesh=_mesh(),
        scratch_types=[
            pltpu.VMEM((CAP * B,), jnp.int32),
            pltpu.VMEM((CAP * B,), jnp.int32),
            pltpu.VMEM((CAP * B,), jnp.int32),
            pltpu.VMEM((2, B, H), jnp.float32),
            pltpu.VMEM((B,), jnp.int32),
            pltpu.VMEM((B, H), jnp.float32),
            pltpu.SemaphoreType.DMA,
            pltpu.SemaphoreType.DMA,
            pltpu.SemaphoreType.DMA,
        ],
    )
    def k(inc, bh, src, rs, rl, out, sidx_a, rs_a, rl_a, gbuf, ridx, rbuf,
          s0, s1, sem2):
        c = lax.axis_index("c")
        s = lax.axis_index("s")
        w = s * _NSC + c
        lo = (NB * w) // _NW
        hi = (NB * (w + 1)) // _NW
        nr = hi - lo
        sems = [s0, s1]

        pltpu.sync_copy(src.at[pl.ds(lo * B, CAP * B)], sidx_a)
        pltpu.sync_copy(rs.at[pl.ds(lo * B, CAP * B)], rs_a)
        pltpu.sync_copy(rl.at[pl.ds(lo * B, CAP * B)], rl_a)

        def start(j, b):
            pltpu.async_copy(inc.at[sidx_a.at[pl.ds(j * B, B)]], gbuf.at[b],
                             sems[b])

        def wait(b):
            pltpu.make_async_copy(inc.at[sidx_a.at[pl.ds(0, B)]], gbuf.at[b],
                                  sems[b]).wait()

        for b in range(2):
            @pl.when(b < nr)
            def _(b=b):
                start(jnp.int32(b), b)

        def kbody(kk, carry):
            for b in range(2):
                j = kk * 2 + b

                @pl.when(j < nr)
                def _(j=j, b=b):
                    wait(b)

                    mxv = rl_a[pl.ds(j * B, 16)]
                    for g in range(1, B // 16):
                        mxv = jnp.maximum(mxv, rl_a[pl.ds(j * B + g * 16, 16)])
                    mx = mxv[0]
                    for lane in range(1, 16):
                        mx = jnp.maximum(mx, mxv[lane])

                    def rnd(t, c1):
                        for g in range(B // 16):
                            sl16 = pl.ds(j * B + g * 16, 16)
                            rsv = rs_a[sl16]
                            rlv = rl_a[sl16]
                            ridx[pl.ds(g * 16, 16)] = jnp.where(
                                t < rlv, rsv + t, jnp.int32(E))
                        pltpu.async_copy(bh.at[ridx], rbuf, sem2).wait()

                        def sub_row(e, c2):
                            for kk2 in range(H // 16):
                                sl = pl.ds(kk2 * 16, 16)
                                gbuf[b, e, sl] = gbuf[b, e, sl] - rbuf[e, sl]
                            return c2

                        return lax.fori_loop(0, B, sub_row, c1)

                    lax.fori_loop(0, mx, rnd, 0)
                    pltpu.sync_copy(gbuf.at[b], out.at[pl.ds((lo + j) * B, B)])

                    @pl.when(j + 2 < nr)
                    def _():
                        start(j + 2, b)

            return carry

        lax.fori_loop(0, (nr + 1) // 2, kbody, 0)

    return k


@functools.lru_cache(maxsize=None)
def _tc_init(E, EP, DP, H, BE=1600):
    ncomp = E // BE

    def body(ea_ref, w_ref, b_ref, out_ref):
        i = pl.program_id(0)
        v = jnp.maximum(
            jnp.dot(ea_ref[...], w_ref[...], preferred_element_type=jnp.float32)
            + b_ref[...],
            0.0,
        )
        out_ref[...] = jnp.where(i >= ncomp, 0.0, v)

    return pl.pallas_call(
        body,
        grid=(EP // BE,),
        in_specs=[
            pl.BlockSpec((BE, DP), lambda i: (jnp.minimum(i, ncomp - 1), 0)),
            pl.BlockSpec((DP, H), lambda i: (0, 0)),
            pl.BlockSpec((1, H), lambda i: (0, 0)),
        ],
        out_specs=pl.BlockSpec((BE, H), lambda i: (i, 0)),
        out_shape=jax.ShapeDtypeStruct((EP, H), jnp.float32),
    )


@functools.lru_cache(maxsize=None)
def _tc_mlp(E, EP, H, BE=1600):
    ncomp = E // BE

    def body(bh_ref, agg_ref, w1a_ref, w1b_ref, b1_ref, w2_ref, b2_ref, out_ref):
        i = pl.program_id(0)
        bh = bh_ref[...]
        bf = jnp.bfloat16
        h1 = jnp.maximum(
            jnp.dot(bh.astype(bf), w1a_ref[...].astype(bf),
                    preferred_element_type=jnp.float32)
            + jnp.dot(agg_ref[...].astype(bf), w1b_ref[...].astype(bf),
                      preferred_element_type=jnp.float32)
            + b1_ref[...],
            0.0,
        )
        h2 = jnp.maximum(
            jnp.dot(h1.astype(bf), w2_ref[...].astype(bf),
                    preferred_element_type=jnp.float32) + b2_ref[...],
            0.0,
        )
        out_ref[...] = jnp.where(i >= ncomp, 0.0, h2 + bh)

    return pl.pallas_call(
        body,
        grid=(EP // BE,),
        in_specs=[
            pl.BlockSpec((BE, H), lambda i: (i, 0)),
            pl.BlockSpec((BE, H), lambda i: (jnp.minimum(i, ncomp - 1), 0)),
            pl.BlockSpec((H, H), lambda i: (0, 0)),
            pl.BlockSpec((H, H), lambda i: (0, 0)),
            pl.BlockSpec((1, H), lambda i: (0, 0)),
            pl.BlockSpec((H, H), lambda i: (0, 0)),
            pl.BlockSpec((1, H), lambda i: (0, 0)),
        ],
        out_specs=pl.BlockSpec((BE, H), lambda i: (i, 0)),
        out_shape=jax.ShapeDtypeStruct((EP, H), jnp.float32),
    )


@functools.lru_cache(maxsize=None)
def _tc_readout(N, DX, H, BN=2000):
    def body(x_ref, m_ref, wax_ref, wam_ref, b1_ref, w2_ref, b2_ref, out_ref):
        h = jnp.maximum(
            jnp.dot(x_ref[...], wax_ref[...], preferred_element_type=jnp.float32)
            + jnp.dot(m_ref[...], wam_ref[...], preferred_element_type=jnp.float32)
            + b1_ref[...],
            0.0,
        )
        out_ref[...] = (
            jnp.dot(h, w2_ref[...], preferred_element_type=jnp.float32) + b2_ref[...]
        )

    return pl.pallas_call(
        body,
        grid=(N // BN,),
        in_specs=[
            pl.BlockSpec((BN, DX), lambda i: (i, 0)),
            pl.BlockSpec((BN, H), lambda i: (i, 0)),
            pl.BlockSpec((DX, H), lambda i: (0, 0)),
            pl.BlockSpec((H, H), lambda i: (0, 0)),
            pl.BlockSpec((1, H), lambda i: (0, 0)),
            pl.BlockSpec((H, H), lambda i: (0, 0)),
            pl.BlockSpec((1, H), lambda i: (0, 0)),
        ],
        out_specs=pl.BlockSpec((BN, H), lambda i: (i, 0)),
        out_shape=jax.ShapeDtypeStruct((N, H), jnp.float32),
    )


def kernel(x, edge_index, edge_attr, batch, W_ei, b_ei, W_m1, b_m1, W_m2, b_m2,
           W_a1, b_a1, W_a2, b_a2):
    N, DX = x.shape
    E = edge_index.shape[1]
    DE = edge_attr.shape[1]
    H = W_ei.shape[1]
    NUM_STEPS = 3
    DP = 128

    src = edge_index[0]
    tgt = edge_index[1]
    keys = tgt * N + src
    skeys, order = lax.sort_key_val(keys, lax.iota(jnp.int32, E), is_stable=False)
    tgt_s = skeys // N
    src_s = skeys - tgt_s * N
    q = src_s * N + tgt_s
    rs = jnp.searchsorted(skeys, q, side="left").astype(jnp.int32)
    re_ = jnp.searchsorted(skeys, q, side="right").astype(jnp.int32)
    rl = re_ - rs

    W1a = W_m1[:H]
    W1b = W_m1[H:]
    Wax = W_a1[:DX]
    Wam = W_a1[DX:]
    W_ei_p = jnp.zeros((DP, H), jnp.float32).at[:DE].set(W_ei)
    b_ei2 = b_ei.reshape(1, H)
    b_m12 = b_m1.reshape(1, H)
    b_m22 = b_m2.reshape(1, H)
    b_a12 = b_a1.reshape(1, H)
    b_a22 = b_a2.reshape(1, H)

    ea_pad = jnp.pad(edge_attr, ((0, 0), (0, DP - DE)))
    ea_p = _sc_gather_rows(E, DP)(ea_pad, order)

    EP = E + 1600
    bh = _tc_init(E, EP, DP, H)(ea_p, W_ei_p, b_ei2)

    segsum = _sc_segsum(E, N, H)
    aggk = _sc_agg(E, EP, N, H)
    mlp = _tc_mlp(E, EP, H)
    for _ in range(NUM_STEPS):
        inc = segsum(bh, tgt_s)
        agg = aggk(inc, bh, src_s, rs, rl)
        bh = mlp(bh, agg, W1a, W1b, b_m12, W_m2, b_m22)

    msg = segsum(bh, tgt_s)
    return _tc_readout(N, DX, H)(x, msg, Wax, Wam, b_a12, W_a2, b_a22)


# speedup vs baseline: 1.4445x; 1.4445x over previous
import functools

import jax
import jax.numpy as jnp
from jax import lax
from jax.experimental import pallas as pl
from jax.experimental.pallas import tpu as pltpu
from jax.experimental.pallas import tpu_sc as plsc

_NSC = 2
_NTEC = 16
_NW = _NSC * _NTEC


def _mesh():
    return plsc.VectorSubcoreMesh(core_axis_name="c", subcore_axis_name="s")


@functools.lru_cache(maxsize=None)
def _sc_gather_rows(E, D):
    B = 64
    NB = E // B

    @functools.partial(
        pl.kernel,
        out_type=jax.ShapeDtypeStruct((E, D), jnp.float32),
        mesh=_mesh(),
        scratch_types=[
            pltpu.VMEM((B,), jnp.int32),
            pltpu.VMEM((B, D), jnp.float32),
            pltpu.SemaphoreType.DMA,
        ],
    )
    def k(table, idx, out, idx_v, buf, sem):
        c = lax.axis_index("c")
        s = lax.axis_index("s")
        w = s * _NSC + c
        lo = (NB * w) // _NW
        hi = (NB * (w + 1)) // _NW

        def body(r, carry):
            pltpu.sync_copy(idx.at[pl.ds(r * B, B)], idx_v)
            pltpu.async_copy(table.at[idx_v], buf, sem).wait()
            pltpu.sync_copy(buf, out.at[pl.ds(r * B, B)])
            return carry

        lax.fori_loop(lo, hi, body, 0)

    return k


@functools.lru_cache(maxsize=None)
def _sc_segsum(E, N, H):
    B = 128
    NB = E // B
    NCH = H // 128
    CPS = NCH // _NSC
    ZR = 40
    NGB = N // ZR

    @functools.partial(
        pl.kernel,
        out_type=jax.ShapeDtypeStruct((N, H), jnp.float32),
        mesh=_mesh(),
        scratch_types=[
            pltpu.VMEM((B, 128), jnp.float32),
            pltpu.VMEM((B,), jnp.int32),
            pltpu.VMEM((ZR, 128), jnp.float32),
            pltpu.VMEM((ZR, 128), jnp.float32),
            pltpu.VMEM_SHARED((N, 128), jnp.float32),
        ],
    )
    def k(bh, tgt, out, data_v, tidx, zbuf, wbuf, table):
        c = lax.axis_index("c")
        s = lax.axis_index("s")
        zero = jnp.zeros((16,), jnp.float32)

        def zrow(i, carry):
            for j in range(8):
                zbuf[i, pl.ds(j * 16, 16)] = zero
            return carry

        lax.fori_loop(0, ZR, zrow, 0)

        lo = (NB * s) // _NTEC
        hi = (NB * (s + 1)) // _NTEC
        glo = (NGB * s) // _NTEC
        ghi = (NGB * (s + 1)) // _NTEC

        for ci in range(CPS):
            fo = (c * CPS + ci) * 128

            def zblk(g, carry):
                pltpu.sync_copy(zbuf, table.at[pl.ds(g * ZR, ZR)])
                return carry

            lax.fori_loop(glo, ghi, zblk, 0)
            plsc.subcore_barrier()

            def body(r, carry):
                pltpu.sync_copy(tgt.at[pl.ds(r * B, B)], tidx)
                pltpu.sync_copy(bh.at[pl.ds(r * B, B), pl.ds(fo, 128)], data_v)
                pltpu.sync_copy(data_v, table.at[tidx], add=True)
                return carry

            lax.fori_loop(lo, hi, body, 0)
            plsc.subcore_barrier()

            def wblk(g, carry):
                pltpu.sync_copy(table.at[pl.ds(g * ZR, ZR)], wbuf)
                pltpu.sync_copy(wbuf, out.at[pl.ds(g * ZR, ZR), pl.ds(fo, 128)])
                return carry

            lax.fori_loop(glo, ghi, wblk, 0)
            plsc.subcore_barrier()

    return k


@functools.lru_cache(maxsize=None)
def _sc_agg(E, EP, N, H):
    B = 64
    NB = E // B
    CAP = NB // _NW + 1

    @functools.partial(
        pl.kernel,
        out_type=jax.ShapeDtypeStruct((E, H), jnp.float32),
        mesh=_mesh(),
        scratch_types=[
            pltpu.VMEM((CAP * B,), jnp.int32),
            pltpu.VMEM((CAP * B,), jnp.int32),
            pltpu.VMEM((CAP * B,), jnp.int32),
            pltpu.VMEM((2, B, H), jnp.float32),
            pltpu.VMEM((B,), jnp.int32),
            pltpu.VMEM((B, H), jnp.float32),
            pltpu.SemaphoreType.DMA,
            pltpu.SemaphoreType.DMA,
            pltpu.SemaphoreType.DMA,
        ],
    )
    def k(inc, bh, src, rs, rl, out, sidx_a, rs_a, rl_a, gbuf, ridx, rbuf,
          s0, s1, sem2):
        c = lax.axis_index("c")
        s = lax.axis_index("s")
        w = s * _NSC + c
        lo = (NB * w) // _NW
        hi = (NB * (w + 1)) // _NW
        nr = hi - lo
        sems = [s0, s1]

        pltpu.sync_copy(src.at[pl.ds(lo * B, CAP * B)], sidx_a)
        pltpu.sync_copy(rs.at[pl.ds(lo * B, CAP * B)], rs_a)
        pltpu.sync_copy(rl.at[pl.ds(lo * B, CAP * B)], rl_a)

        def start(j, b):
            pltpu.async_copy(inc.at[sidx_a.at[pl.ds(j * B, B)]], gbuf.at[b],
                             sems[b])

        def wait(b):
            pltpu.make_async_copy(inc.at[sidx_a.at[pl.ds(0, B)]], gbuf.at[b],
                                  sems[b]).wait()

        for b in range(2):
            @pl.when(b < nr)
            def _(b=b):
                start(jnp.int32(b), b)

        def kbody(kk, carry):
            for b in range(2):
                j = kk * 2 + b

                @pl.when(j < nr)
                def _(j=j, b=b):
                    wait(b)

                    mxv = rl_a[pl.ds(j * B, 16)]
                    for g in range(1, B // 16):
                        mxv = jnp.maximum(mxv, rl_a[pl.ds(j * B + g * 16, 16)])
                    mx = mxv[0]
                    for lane in range(1, 16):
                        mx = jnp.maximum(mx, mxv[lane])

                    def rnd(t, c1):
                        for g in range(B // 16):
                            sl16 = pl.ds(j * B + g * 16, 16)
                            rsv = rs_a[sl16]
                            rlv = rl_a[sl16]
                            ridx[pl.ds(g * 16, 16)] = jnp.where(
                                t < rlv, rsv + t, jnp.int32(E))
                        pltpu.async_copy(bh.at[ridx], rbuf, sem2).wait()

                        def sub_row(e, c2):
                            for kk2 in range(H // 16):
                                sl = pl.ds(kk2 * 16, 16)
                                gbuf[b, e, sl] = gbuf[b, e, sl] - rbuf[e, sl]
                            return c2

                        return lax.fori_loop(0, B, sub_row, c1)

                    lax.fori_loop(0, mx, rnd, 0)
                    pltpu.sync_copy(gbuf.at[b], out.at[pl.ds((lo + j) * B, B)])

                    @pl.when(j + 2 < nr)
                    def _():
                        start(j + 2, b)

            return carry

        lax.fori_loop(0, (nr + 1) // 2, kbody, 0)

    return k


@functools.lru_cache(maxsize=None)
def _tc_init(E, EP, DP, H, BE=1600):
    ncomp = E // BE

    def body(ea_ref, w_ref, b_ref, out_ref):
        i = pl.program_id(0)
        v = jnp.maximum(
            jnp.dot(ea_ref[...], w_ref[...], preferred_element_type=jnp.float32)
            + b_ref[...],
            0.0,
        )
        out_ref[...] = jnp.where(i >= ncomp, 0.0, v)

    return pl.pallas_call(
        body,
        grid=(EP // BE,),
        in_specs=[
            pl.BlockSpec((BE, DP), lambda i: (jnp.minimum(i, ncomp - 1), 0)),
            pl.BlockSpec((DP, H), lambda i: (0, 0)),
            pl.BlockSpec((1, H), lambda i: (0, 0)),
        ],
        out_specs=pl.BlockSpec((BE, H), lambda i: (i, 0)),
        out_shape=jax.ShapeDtypeStruct((EP, H), jnp.float32),
    )


@functools.lru_cache(maxsize=None)
def _tc_mlp(E, EP, H, BE=1600):
    ncomp = E // BE

    def body(bh_ref, agg_ref, w1a_ref, w1b_ref, b1_ref, w2_ref, b2_ref, out_ref):
        i = pl.program_id(0)
        bh = bh_ref[...]
        bf = jnp.bfloat16
        h1 = jnp.maximum(
            jnp.dot(bh.astype(bf), w1a_ref[...].astype(bf),
                    preferred_element_type=jnp.float32)
            + jnp.dot(agg_ref[...].astype(bf), w1b_ref[...].astype(bf),
                      preferred_element_type=jnp.float32)
            + b1_ref[...],
            0.0,
        )
        h2 = jnp.maximum(
            jnp.dot(h1.astype(bf), w2_ref[...].astype(bf),
                    preferred_element_type=jnp.float32) + b2_ref[...],
            0.0,
        )
        out_ref[...] = jnp.where(i >= ncomp, 0.0, h2 + bh)

    return pl.pallas_call(
        body,
        grid=(EP // BE,),
        in_specs=[
            pl.BlockSpec((BE, H), lambda i: (i, 0)),
            pl.BlockSpec((BE, H), lambda i: (jnp.minimum(i, ncomp - 1), 0)),
            pl.BlockSpec((H, H), lambda i: (0, 0)),
            pl.BlockSpec((H, H), lambda i: (0, 0)),
            pl.BlockSpec((1, H), lambda i: (0, 0)),
            pl.BlockSpec((H, H), lambda i: (0, 0)),
            pl.BlockSpec((1, H), lambda i: (0, 0)),
        ],
        out_specs=pl.BlockSpec((BE, H), lambda i: (i, 0)),
        out_shape=jax.ShapeDtypeStruct((EP, H), jnp.float32),
    )


@functools.lru_cache(maxsize=None)
def _tc_readout(N, DX, H, BN=2000):
    def body(x_ref, m_ref, wax_ref, wam_ref, b1_ref, w2_ref, b2_ref, out_ref):
        h = jnp.maximum(
            jnp.dot(x_ref[...], wax_ref[...], preferred_element_type=jnp.float32)
            + jnp.dot(m_ref[...], wam_ref[...], preferred_element_type=jnp.float32)
            + b1_ref[...],
            0.0,
        )
        out_ref[...] = (
            jnp.dot(h, w2_ref[...], preferred_element_type=jnp.float32) + b2_ref[...]
        )

    return pl.pallas_call(
        body,
        grid=(N // BN,),
        in_specs=[
            pl.BlockSpec((BN, DX), lambda i: (i, 0)),
            pl.BlockSpec((BN, H), lambda i: (i, 0)),
            pl.BlockSpec((DX, H), lambda i: (0, 0)),
            pl.BlockSpec((H, H), lambda i: (0, 0)),
            pl.BlockSpec((1, H), lambda i: (0, 0)),
            pl.BlockSpec((H, H), lambda i: (0, 0)),
            pl.BlockSpec((1, H), lambda i: (0, 0)),
        ],
        out_specs=pl.BlockSpec((BN, H), lambda i: (i, 0)),
        out_shape=jax.ShapeDtypeStruct((N, H), jnp.float32),
    )


def kernel(x, edge_index, edge_attr, batch, W_ei, b_ei, W_m1, b_m1, W_m2, b_m2,
           W_a1, b_a1, W_a2, b_a2):
    N, DX = x.shape
    E = edge_index.shape[1]
    DE = edge_attr.shape[1]
    H = W_ei.shape[1]
    NUM_STEPS = 3
    DP = 128

    src = edge_index[0]
    tgt = edge_index[1]
    keys = tgt * N + src
    skeys, order = lax.sort_key_val(keys, lax.iota(jnp.int32, E), is_stable=False)
    tgt_s = skeys // N
    src_s = skeys - tgt_s * N
    q = src_s * N + tgt_s
    iot = lax.iota(jnp.int32, E)
    cv = jnp.concatenate([q * 4, q * 4 + 3, skeys * 4 + 1])
    cp = jnp.concatenate([iot, E + iot, 2 * E + iot])
    sv, sp = lax.sort_key_val(cv, cp, is_stable=False)
    nk = jnp.cumsum((sp >= 2 * E).astype(jnp.int32))
    _, back = lax.sort_key_val(sp, nk, is_stable=False)
    rs = back[:E]
    rl = back[E:2 * E] - rs

    W1a = W_m1[:H]
    W1b = W_m1[H:]
    Wax = W_a1[:DX]
    Wam = W_a1[DX:]
    W_ei_p = jnp.zeros((DP, H), jnp.float32).at[:DE].set(W_ei)
    b_ei2 = b_ei.reshape(1, H)
    b_m12 = b_m1.reshape(1, H)
    b_m22 = b_m2.reshape(1, H)
    b_a12 = b_a1.reshape(1, H)
    b_a22 = b_a2.reshape(1, H)

    ea_pad = jnp.pad(edge_attr, ((0, 0), (0, DP - DE)))
    ea_p = _sc_gather_rows(E, DP)(ea_pad, order)

    EP = E + 1600
    bh = _tc_init(E, EP, DP, H)(ea_p, W_ei_p, b_ei2)

    segsum = _sc_segsum(E, N, H)
    aggk = _sc_agg(E, EP, N, H)
    mlp = _tc_mlp(E, EP, H)
    for _ in range(NUM_STEPS):
        inc = segsum(bh, tgt_s)
        agg = aggk(inc, bh, src_s, rs, rl)
        bh = mlp(bh, agg, W1a, W1b, b_m12, W_m2, b_m22)

    msg = segsum(bh, tgt_s)
    return _tc_readout(N, DX, H)(x, msg, Wax, Wam, b_a12, W_a2, b_a22)
